# baseline (speedup 1.0000x reference)
import functools
import jax
import jax.numpy as jnp
from jax import lax
from jax.experimental import pallas as pl
from jax.experimental.pallas import tpu as pltpu
from jax.experimental.pallas import tpu_sc as plsc

B = 16384
K = 200
NC = 2
NS = 16
NW = NC * NS
RPW = B // NW
NG = RPW // 16
CH = 128
NCK = RPW // CH
GPC = CH // 16
UNROLL = 8
NCHUNK = K // UNROLL
EPS = 1e-7
LN2 = 0.6931471805599453
MASK23 = 0x007FFFFF
ONEBITS = 0x3F800000
C0 = 2.885390081777927
C1 = 0.961796693925976
C2 = 0.5770780163555854
C3 = 0.41219858311113246
C4 = 0.32059889797532526


def _log2_mant(m):
    u = (m - 1.0) / (m + 1.0)
    u2 = u * u
    return u * (C0 + u2 * (C1 + u2 * (C2 + u2 * (C3 + u2 * C4))))


def _ln(t):
    bits = plsc.bitcast(t, jnp.int32)
    e = (bits >> 23) - 127
    m = plsc.bitcast((bits & MASK23) | ONEBITS, jnp.float32)
    return (e.astype(jnp.float32) + _log2_mant(m)) * LN2


def _sc_body(pred_hbm, edges_hbm, dur_hbm, ev_hbm, out_hbm,
             pred_v0, pred_v1, edges_v, dur_v, ev_v, stage_v, sem0, sem1):
    wid = lax.axis_index("s") * NC + lax.axis_index("c")
    base = wid * RPW
    pltpu.sync_copy(edges_hbm, edges_v)
    pltpu.sync_copy(dur_hbm.at[pl.ds(base, RPW)], dur_v)
    pltpu.sync_copy(ev_hbm.at[pl.ds(base, RPW)], ev_v)

    lanes = lax.iota(jnp.int32, 16)
    inv_step = edges_v[pl.ds(208, 16)]

    def make_group_body(ck, pred_v):
      def group_body(gg, carry):
        acc_s, acc_h, in01_f, dec_f = carry
        g = ck * GPC + gg
        d = dur_v[pl.ds(g * 16, 16)]
        evv = ev_v[pl.ds(g * 16, 16)]
        is_ev = evv != 0

        a = d * inv_step
        c = a.astype(jnp.int32)
        bb = jnp.clip(c - 1, 0, K - 3)
        p = bb
        for t in range(4):
            ec = plsc.load_gather(edges_v, [jnp.minimum(bb + t, K)])
            p = p + jnp.where(ec < d, 1, 0).astype(jnp.int32)
        idx = jnp.clip(p - 1, 0, K - 1)

        local_rows = gg * 16 + lanes
        zcol = lanes * 0

        def chunk_body(jj, ch):
            (e_s, m_s0, m_s1, e_h, m_h0, m_h1, prev_x, s_prev,
             dmin0, dmin1) = ch
            j0 = jj * UNROLL
            for dj in range(UNROLL):
                j = j0 + dj
                x = plsc.load_gather(pred_v, [local_rows, zcol + j])
                if dj % 2 == 0:
                    dmin0 = jnp.minimum(dmin0, prev_x - x)
                else:
                    dmin1 = jnp.minimum(dmin1, prev_x - x)
                prev_x = x
                m_lt = j < idx
                t_h = jnp.clip(1.0 - x, EPS, 1.0 - EPS)
                t_h = jnp.where(m_lt, t_h, 1.0)
                tb = plsc.bitcast(t_h, jnp.int32)
                e_h = e_h + (tb >> 23)
                mant = plsc.bitcast((tb & MASK23) | ONEBITS, jnp.float32)
                if dj % 2 == 0:
                    m_h0 = m_h0 * mant
                else:
                    m_h1 = m_h1 * mant
                s = jnp.maximum(x, EPS)
                t_s = jnp.minimum(s / s_prev, 1.0 - EPS)
                s_prev = s
                t_s = jnp.where(m_lt, t_s, 1.0)
                tb = plsc.bitcast(t_s, jnp.int32)
                e_s = e_s + (tb >> 23)
                mant = plsc.bitcast((tb & MASK23) | ONEBITS, jnp.float32)
                if dj % 2 == 0:
                    m_s0 = m_s0 * mant
                else:
                    m_s1 = m_s1 * mant
            mb = plsc.bitcast(m_s0, jnp.int32)
            e_s = e_s + (mb >> 23)
            m_s0 = plsc.bitcast((mb & MASK23) | ONEBITS, jnp.float32)
            mb = plsc.bitcast(m_s1, jnp.int32)
            e_s = e_s + (mb >> 23)
            m_s1 = plsc.bitcast((mb & MASK23) | ONEBITS, jnp.float32)
            mb = plsc.bitcast(m_h0, jnp.int32)
            e_h = e_h + (mb >> 23)
            m_h0 = plsc.bitcast((mb & MASK23) | ONEBITS, jnp.float32)
            mb = plsc.bitcast(m_h1, jnp.int32)
            e_h = e_h + (mb >> 23)
            m_h1 = plsc.bitcast((mb & MASK23) | ONEBITS, jnp.float32)
            return (e_s, m_s0, m_s1, e_h, m_h0, m_h1, prev_x, s_prev,
                    dmin0, dmin1)

        zi = lanes * 0
        zf = zi.astype(jnp.float32)
        init = (zi, zf + 1.0, zf + 1.0, zi, zf + 1.0, zf + 1.0,
                zf + 3e38, zf + 1.0, zf + 3e38, zf + 3e38)
        (e_s, m_s0, m_s1, e_h, m_h0, m_h1, _, _,
         dmin0, dmin1) = lax.fori_loop(0, NCHUNK, chunk_body, init)
        dmin = jnp.minimum(dmin0, dmin1)
        dec_f = jnp.minimum(dec_f, jnp.where(dmin >= -1e-6, 1.0, 0.0))
        mb = plsc.bitcast(m_s0 * m_s1, jnp.int32)
        e_s = e_s + (mb >> 23)
        m_s = plsc.bitcast((mb & MASK23) | ONEBITS, jnp.float32)
        mb = plsc.bitcast(m_h0 * m_h1, jnp.int32)
        e_h = e_h + (mb >> 23)
        m_h = plsc.bitcast((mb & MASK23) | ONEBITS, jnp.float32)

        x_at = plsc.load_gather(pred_v, [local_rows, idx])
        x_pv = plsc.load_gather(pred_v, [local_rows, jnp.maximum(idx - 1, 0)])
        h_h_at = jnp.clip(x_at, EPS, 1.0 - EPS)
        s_at = jnp.clip(x_at, EPS, 1.0)
        s_pv = jnp.where(idx == 0, 1.0, jnp.clip(x_pv, EPS, 1.0))
        h_s_at = jnp.clip(1.0 - s_at / s_pv, EPS, 1.0 - EPS)

        ebias = 127 * (K + 2 * NCHUNK + 1)
        prefix_s = ((e_s - ebias).astype(jnp.float32)
                    + _log2_mant(m_s)) * LN2
        tail_s = jnp.where(is_ev, h_s_at, 1.0 - h_s_at)
        ll_s = prefix_s + _ln(tail_s)
        fin_s = (ll_s > -1e30) & (ll_s < 1e30)
        acc_s = acc_s + jnp.where(fin_s, ll_s, -1e6)

        prefix_h = ((e_h - ebias).astype(jnp.float32)
                    + _log2_mant(m_h)) * LN2
        tail_h = jnp.where(is_ev, h_h_at, 1.0 - h_h_at)
        ll_h = prefix_h + _ln(tail_h)
        fin_h = (ll_h > -1e30) & (ll_h < 1e30)
        acc_h = acc_h + jnp.where(fin_h, ll_h, -1e6)

        return (acc_s, acc_h, in01_f, dec_f)
      return group_body

    zf = lanes.astype(jnp.float32) * 0.0
    carry = (zf, zf, zf + 1.0, zf + 1.0)
    bufs = [pred_v0, pred_v1]
    sems = [sem0, sem1]
    copies = [None, None]
    copies[0] = pltpu.async_copy(
        pred_hbm.at[pl.ds(base, CH)], bufs[0], sems[0])
    for ck in range(NCK):
        if ck + 1 < NCK:
            copies[(ck + 1) % 2] = pltpu.async_copy(
                pred_hbm.at[pl.ds(base + (ck + 1) * CH, CH)],
                bufs[(ck + 1) % 2], sems[(ck + 1) % 2])
        copies[ck % 2].wait()
        carry = lax.fori_loop(0, GPC, make_group_body(ck, bufs[ck % 2]), carry)
    acc_s, acc_h, in01_f, dec_f = carry

    stage_v[pl.ds(0, 16)] = acc_s
    stage_v[pl.ds(16, 16)] = acc_h
    stage_v[pl.ds(32, 16)] = in01_f
    stage_v[pl.ds(48, 16)] = dec_f
    pltpu.sync_copy(stage_v, out_hbm.at[wid])


@jax.jit
def kernel(pred_prob, true_time, true_event):
    pred = pred_prob.astype(jnp.float32)
    dur = true_time.astype(jnp.float32).reshape(-1)
    ev = true_event.reshape(-1).astype(jnp.int32)
    max_t = jnp.clip(jnp.max(dur), 1e-6, None)
    edges = jnp.linspace(0.0, max_t, K + 1).astype(jnp.float32)
    edges_pad = jnp.zeros((224,), jnp.float32)
    edges_pad = edges_pad.at[:K + 1].set(edges)
    edges_pad = edges_pad.at[208:].set(jnp.float32(K) / max_t)

    mesh = plsc.VectorSubcoreMesh(core_axis_name="c", subcore_axis_name="s",
                                  num_cores=NC, num_subcores=NS)
    run = pl.kernel(
        _sc_body,
        out_type=jax.ShapeDtypeStruct((NW, 64), jnp.float32),
        mesh=mesh,
        compiler_params=pltpu.CompilerParams(needs_layout_passes=False),
        scratch_types=[
            pltpu.VMEM((CH, K), jnp.float32),
            pltpu.VMEM((CH, K), jnp.float32),
            pltpu.VMEM((224,), jnp.float32),
            pltpu.VMEM((RPW,), jnp.float32),
            pltpu.VMEM((RPW,), jnp.int32),
            pltpu.VMEM((64,), jnp.float32),
            pltpu.SemaphoreType.DMA,
            pltpu.SemaphoreType.DMA,
        ],
    )
    parts = run(pred, edges_pad, dur, ev)

    sum_s = jnp.sum(parts[:, 0:16])
    sum_h = jnp.sum(parts[:, 16:32])
    cond = (jnp.min(parts[:, 32:48]) > 0.5) & (jnp.min(parts[:, 48:64]) > 0.5)
    return -jnp.where(cond, sum_s, sum_h) / B


# speedup vs baseline: 1.9674x; 1.9674x over previous
import functools
import jax
import jax.numpy as jnp
from jax import lax
from jax.experimental import pallas as pl
from jax.experimental.pallas import tpu as pltpu
from jax.experimental.pallas import tpu_sc as plsc

B = 16384
K = 200
NC = 2
NS = 16
NW = NC * NS
RPW = B // NW
NG = RPW // 16
UNROLL = 8
NCHUNK = K // UNROLL
EPS = 1e-7
LN2 = 0.6931471805599453
MASK23 = 0x007FFFFF
ONEBITS = 0x3F800000
C0 = 2.885390081777927
C1 = 0.961796693925976
C2 = 0.5770780163555854
C3 = 0.41219858311113246
C4 = 0.32059889797532526


def _log2_mant(m):
    u = (m - 1.0) / (m + 1.0)
    u2 = u * u
    return u * (C0 + u2 * (C1 + u2 * (C2 + u2 * (C3 + u2 * C4))))


def _ln(t):
    bits = plsc.bitcast(t, jnp.int32)
    e = (bits >> 23) - 127
    m = plsc.bitcast((bits & MASK23) | ONEBITS, jnp.float32)
    return (e.astype(jnp.float32) + _log2_mant(m)) * LN2


def _sc_body(predt_hbm, edges_hbm, dur_hbm, ev_hbm, out_hbm,
             pred_v, edges_v, dur_v, ev_v, stage_v):
    wid = lax.axis_index("s") * NC + lax.axis_index("c")
    base = wid * RPW
    pltpu.sync_copy(predt_hbm.at[:, pl.ds(base, RPW)], pred_v)
    pltpu.sync_copy(edges_hbm, edges_v)
    pltpu.sync_copy(dur_hbm.at[pl.ds(base, RPW)], dur_v)
    pltpu.sync_copy(ev_hbm.at[pl.ds(base, RPW)], ev_v)

    lanes = lax.iota(jnp.int32, 16)
    inv_step = edges_v[pl.ds(208, 16)]

    def group_body(g, carry):
        acc_s, acc_h, in01_f, dec_f = carry
        go = g * 16
        d = dur_v[pl.ds(go, 16)]
        evv = ev_v[pl.ds(go, 16)]
        is_ev = evv != 0

        a = d * inv_step
        c = a.astype(jnp.int32)
        bb = jnp.clip(c - 1, 0, K - 3)
        p = bb
        for t in range(4):
            ec = plsc.load_gather(edges_v, [jnp.minimum(bb + t, K)])
            p = p + jnp.where(ec < d, 1, 0).astype(jnp.int32)
        idx = jnp.clip(p - 1, 0, K - 1)

        def chunk_body(jj, ch):
            (e_s, m_s0, m_s1, e_h, m_h0, m_h1, prev_x, s_prev,
             dmin0, dmin1) = ch
            j0 = jj * UNROLL
            for dj in range(UNROLL):
                j = j0 + dj
                x = pred_v[j, pl.ds(go, 16)]
                if dj % 2 == 0:
                    dmin0 = jnp.minimum(dmin0, prev_x - x)
                else:
                    dmin1 = jnp.minimum(dmin1, prev_x - x)
                prev_x = x
                m_lt = j < idx
                t_h = jnp.clip(1.0 - x, EPS, 1.0 - EPS)
                t_h = jnp.where(m_lt, t_h, 1.0)
                tb = plsc.bitcast(t_h, jnp.int32)
                e_h = e_h + (tb >> 23)
                mant = plsc.bitcast((tb & MASK23) | ONEBITS, jnp.float32)
                if dj % 2 == 0:
                    m_h0 = m_h0 * mant
                else:
                    m_h1 = m_h1 * mant
                s = jnp.maximum(x, EPS)
                t_s = jnp.minimum(s / s_prev, 1.0 - EPS)
                s_prev = s
                t_s = jnp.where(m_lt, t_s, 1.0)
                tb = plsc.bitcast(t_s, jnp.int32)
                e_s = e_s + (tb >> 23)
                mant = plsc.bitcast((tb & MASK23) | ONEBITS, jnp.float32)
                if dj % 2 == 0:
                    m_s0 = m_s0 * mant
                else:
                    m_s1 = m_s1 * mant
            mb = plsc.bitcast(m_s0, jnp.int32)
            e_s = e_s + (mb >> 23)
            m_s0 = plsc.bitcast((mb & MASK23) | ONEBITS, jnp.float32)
            mb = plsc.bitcast(m_s1, jnp.int32)
            e_s = e_s + (mb >> 23)
            m_s1 = plsc.bitcast((mb & MASK23) | ONEBITS, jnp.float32)
            mb = plsc.bitcast(m_h0, jnp.int32)
            e_h = e_h + (mb >> 23)
            m_h0 = plsc.bitcast((mb & MASK23) | ONEBITS, jnp.float32)
            mb = plsc.bitcast(m_h1, jnp.int32)
            e_h = e_h + (mb >> 23)
            m_h1 = plsc.bitcast((mb & MASK23) | ONEBITS, jnp.float32)
            return (e_s, m_s0, m_s1, e_h, m_h0, m_h1, prev_x, s_prev,
                    dmin0, dmin1)

        zi = lanes * 0
        zf = zi.astype(jnp.float32)
        init = (zi, zf + 1.0, zf + 1.0, zi, zf + 1.0, zf + 1.0,
                zf + 3e38, zf + 1.0, zf + 3e38, zf + 3e38)
        (e_s, m_s0, m_s1, e_h, m_h0, m_h1, _, _,
         dmin0, dmin1) = lax.fori_loop(0, NCHUNK, chunk_body, init)
        dmin = jnp.minimum(dmin0, dmin1)
        dec_f = jnp.minimum(dec_f, jnp.where(dmin >= -1e-6, 1.0, 0.0))
        mb = plsc.bitcast(m_s0 * m_s1, jnp.int32)
        e_s = e_s + (mb >> 23)
        m_s = plsc.bitcast((mb & MASK23) | ONEBITS, jnp.float32)
        mb = plsc.bitcast(m_h0 * m_h1, jnp.int32)
        e_h = e_h + (mb >> 23)
        m_h = plsc.bitcast((mb & MASK23) | ONEBITS, jnp.float32)

        cols = go + lanes
        x_at = plsc.load_gather(pred_v, [idx, cols])
        x_pv = plsc.load_gather(pred_v, [jnp.maximum(idx - 1, 0), cols])
        h_h_at = jnp.clip(x_at, EPS, 1.0 - EPS)
        s_at = jnp.clip(x_at, EPS, 1.0)
        s_pv = jnp.where(idx == 0, 1.0, jnp.clip(x_pv, EPS, 1.0))
        h_s_at = jnp.clip(1.0 - s_at / s_pv, EPS, 1.0 - EPS)

        ebias = 127 * (K + 2 * NCHUNK + 1)
        prefix_s = ((e_s - ebias).astype(jnp.float32)
                    + _log2_mant(m_s)) * LN2
        tail_s = jnp.where(is_ev, h_s_at, 1.0 - h_s_at)
        ll_s = prefix_s + _ln(tail_s)
        fin_s = (ll_s > -1e30) & (ll_s < 1e30)
        acc_s = acc_s + jnp.where(fin_s, ll_s, -1e6)

        prefix_h = ((e_h - ebias).astype(jnp.float32)
                    + _log2_mant(m_h)) * LN2
        tail_h = jnp.where(is_ev, h_h_at, 1.0 - h_h_at)
        ll_h = prefix_h + _ln(tail_h)
        fin_h = (ll_h > -1e30) & (ll_h < 1e30)
        acc_h = acc_h + jnp.where(fin_h, ll_h, -1e6)

        return (acc_s, acc_h, in01_f, dec_f)

    zf = lanes.astype(jnp.float32) * 0.0
    acc_s, acc_h, in01_f, dec_f = lax.fori_loop(
        0, NG, group_body, (zf, zf, zf + 1.0, zf + 1.0))

    stage_v[pl.ds(0, 16)] = acc_s
    stage_v[pl.ds(16, 16)] = acc_h
    stage_v[pl.ds(32, 16)] = in01_f
    stage_v[pl.ds(48, 16)] = dec_f
    pltpu.sync_copy(stage_v, out_hbm.at[wid])


@jax.jit
def kernel(pred_prob, true_time, true_event):
    pred = pred_prob.astype(jnp.float32)
    dur = true_time.astype(jnp.float32).reshape(-1)
    ev = true_event.reshape(-1).astype(jnp.int32)
    max_t = jnp.clip(jnp.max(dur), 1e-6, None)
    edges = jnp.linspace(0.0, max_t, K + 1).astype(jnp.float32)
    edges_pad = jnp.zeros((224,), jnp.float32)
    edges_pad = edges_pad.at[:K + 1].set(edges)
    edges_pad = edges_pad.at[208:].set(jnp.float32(K) / max_t)

    mesh = plsc.VectorSubcoreMesh(core_axis_name="c", subcore_axis_name="s",
                                  num_cores=NC, num_subcores=NS)
    run = pl.kernel(
        _sc_body,
        out_type=jax.ShapeDtypeStruct((NW, 64), jnp.float32),
        mesh=mesh,
        compiler_params=pltpu.CompilerParams(needs_layout_passes=False),
        scratch_types=[
            pltpu.VMEM((K, RPW), jnp.float32),
            pltpu.VMEM((224,), jnp.float32),
            pltpu.VMEM((RPW,), jnp.float32),
            pltpu.VMEM((RPW,), jnp.int32),
            pltpu.VMEM((64,), jnp.float32),
        ],
    )
    parts = run(pred.T, edges_pad, dur, ev)

    sum_s = jnp.sum(parts[:, 0:16])
    sum_h = jnp.sum(parts[:, 16:32])
    cond = (jnp.min(parts[:, 32:48]) > 0.5) & (jnp.min(parts[:, 48:64]) > 0.5)
    return -jnp.where(cond, sum_s, sum_h) / B


# speedup vs baseline: 2.0799x; 1.0572x over previous
import functools
import jax
import jax.numpy as jnp
from jax import lax
from jax.experimental import pallas as pl
from jax.experimental.pallas import tpu as pltpu
from jax.experimental.pallas import tpu_sc as plsc

B = 16384
K = 200
NC = 2
NS = 16
NW = NC * NS
RPW = B // NW
NG = RPW // 16
UNROLL = 8
NCHUNK = K // UNROLL
EPS = 1e-7
LN2 = 0.6931471805599453
MASK23 = 0x007FFFFF
ONEBITS = 0x3F800000
C0 = 2.885390081777927
C1 = 0.961796693925976
C2 = 0.5770780163555854
C3 = 0.41219858311113246
C4 = 0.32059889797532526


def _log2_mant(m):
    u = (m - 1.0) / (m + 1.0)
    u2 = u * u
    return u * (C0 + u2 * (C1 + u2 * (C2 + u2 * (C3 + u2 * C4))))


def _ln(t):
    bits = plsc.bitcast(t, jnp.int32)
    e = (bits >> 23) - 127
    m = plsc.bitcast((bits & MASK23) | ONEBITS, jnp.float32)
    return (e.astype(jnp.float32) + _log2_mant(m)) * LN2


def _sc_body(predt_hbm, edges_hbm, dur_hbm, ev_hbm, out_hbm,
             pred_v, edges_v, dur_v, ev_v, stage_v, sem0, sem1):
    wid = lax.axis_index("s") * NC + lax.axis_index("c")
    base = wid * RPW
    half = RPW // 2
    cp0 = pltpu.async_copy(predt_hbm.at[:, pl.ds(base, half)],
                           pred_v.at[:, pl.ds(0, half)], sem0)
    cp1 = pltpu.async_copy(predt_hbm.at[:, pl.ds(base + half, half)],
                           pred_v.at[:, pl.ds(half, half)], sem1)
    pltpu.sync_copy(edges_hbm, edges_v)
    pltpu.sync_copy(dur_hbm.at[pl.ds(base, RPW)], dur_v)
    pltpu.sync_copy(ev_hbm.at[pl.ds(base, RPW)], ev_v)

    lanes = lax.iota(jnp.int32, 16)
    inv_step = edges_v[pl.ds(208, 16)]

    def group_body(g, carry):
        acc_s, acc_h, in01_f, dec_f = carry
        go = g * 16
        d = dur_v[pl.ds(go, 16)]
        evv = ev_v[pl.ds(go, 16)]
        is_ev = evv != 0

        a = d * inv_step
        c = a.astype(jnp.int32)
        bb = jnp.clip(c - 1, 0, K - 3)
        p = bb
        for t in range(4):
            ec = plsc.load_gather(edges_v, [jnp.minimum(bb + t, K)])
            p = p + jnp.where(ec < d, 1, 0).astype(jnp.int32)
        idx = jnp.clip(p - 1, 0, K - 1)

        def chunk_body(jj, ch):
            (e_s, m_s, e_h, m_h, prev_x, s_prev, dmin) = ch
            j0 = jj * UNROLL
            for dj in range(UNROLL):
                j = j0 + dj
                x = pred_v[j, pl.ds(go, 16)]
                dmin = jnp.minimum(dmin, prev_x - x)
                prev_x = x
                m_lt = j < idx
                t_h = jnp.clip(1.0 - x, EPS, 1.0 - EPS)
                t_h = jnp.where(m_lt, t_h, 1.0)
                tb = plsc.bitcast(t_h, jnp.int32)
                e_h = e_h + (tb >> 23)
                m_h = m_h * plsc.bitcast((tb & MASK23) | ONEBITS, jnp.float32)
                s = jnp.maximum(x, EPS)
                t_s = jnp.minimum(s / s_prev, 1.0 - EPS)
                s_prev = s
                t_s = jnp.where(m_lt, t_s, 1.0)
                tb = plsc.bitcast(t_s, jnp.int32)
                e_s = e_s + (tb >> 23)
                m_s = m_s * plsc.bitcast((tb & MASK23) | ONEBITS, jnp.float32)
            mb = plsc.bitcast(m_s, jnp.int32)
            e_s = e_s + (mb >> 23)
            m_s = plsc.bitcast((mb & MASK23) | ONEBITS, jnp.float32)
            mb = plsc.bitcast(m_h, jnp.int32)
            e_h = e_h + (mb >> 23)
            m_h = plsc.bitcast((mb & MASK23) | ONEBITS, jnp.float32)
            return (e_s, m_s, e_h, m_h, prev_x, s_prev, dmin)

        zi = lanes * 0
        zf = zi.astype(jnp.float32)
        init = (zi, zf + 1.0, zi, zf + 1.0, zf + 3e38, zf + 1.0, zf + 3e38)
        (e_s, m_s, e_h, m_h, _, _, dmin) = lax.fori_loop(
            0, NCHUNK, chunk_body, init)
        dec_f = jnp.minimum(dec_f, jnp.where(dmin >= -1e-6, 1.0, 0.0))

        cols = go + lanes
        x_at = plsc.load_gather(pred_v, [idx, cols])
        x_pv = plsc.load_gather(pred_v, [jnp.maximum(idx - 1, 0), cols])
        h_h_at = jnp.clip(x_at, EPS, 1.0 - EPS)
        s_at = jnp.clip(x_at, EPS, 1.0)
        s_pv = jnp.where(idx == 0, 1.0, jnp.clip(x_pv, EPS, 1.0))
        h_s_at = jnp.clip(1.0 - s_at / s_pv, EPS, 1.0 - EPS)

        ebias = 127 * (K + NCHUNK + 2)
        tail_s = jnp.where(is_ev, h_s_at, 1.0 - h_s_at)
        tb = plsc.bitcast(tail_s, jnp.int32)
        e_s = e_s + (tb >> 23)
        mm = m_s * plsc.bitcast((tb & MASK23) | ONEBITS, jnp.float32)
        mb = plsc.bitcast(mm, jnp.int32)
        e_s = e_s + (mb >> 23)
        m_c = plsc.bitcast((mb & MASK23) | ONEBITS, jnp.float32)
        ll_s = ((e_s - ebias).astype(jnp.float32) + _log2_mant(m_c)) * LN2
        fin_s = (ll_s > -1e30) & (ll_s < 1e30)
        acc_s = acc_s + jnp.where(fin_s, ll_s, -1e6)

        tail_h = jnp.where(is_ev, h_h_at, 1.0 - h_h_at)
        tb = plsc.bitcast(tail_h, jnp.int32)
        e_h = e_h + (tb >> 23)
        mm = m_h * plsc.bitcast((tb & MASK23) | ONEBITS, jnp.float32)
        mb = plsc.bitcast(mm, jnp.int32)
        e_h = e_h + (mb >> 23)
        m_c = plsc.bitcast((mb & MASK23) | ONEBITS, jnp.float32)
        ll_h = ((e_h - ebias).astype(jnp.float32) + _log2_mant(m_c)) * LN2
        fin_h = (ll_h > -1e30) & (ll_h < 1e30)
        acc_h = acc_h + jnp.where(fin_h, ll_h, -1e6)

        return (acc_s, acc_h, in01_f, dec_f)

    zf = lanes.astype(jnp.float32) * 0.0
    carry = (zf, zf, zf + 1.0, zf + 1.0)
    cp0.wait()
    carry = lax.fori_loop(0, NG // 2, group_body, carry)
    cp1.wait()
    carry = lax.fori_loop(NG // 2, NG, group_body, carry)
    acc_s, acc_h, in01_f, dec_f = carry

    stage_v[pl.ds(0, 16)] = acc_s
    stage_v[pl.ds(16, 16)] = acc_h
    stage_v[pl.ds(32, 16)] = in01_f
    stage_v[pl.ds(48, 16)] = dec_f
    pltpu.sync_copy(stage_v, out_hbm.at[wid])


@jax.jit
def kernel(pred_prob, true_time, true_event):
    pred = pred_prob.astype(jnp.float32)
    dur = true_time.astype(jnp.float32).reshape(-1)
    ev = true_event.reshape(-1).astype(jnp.int32)
    max_t = jnp.clip(jnp.max(dur), 1e-6, None)
    edges = jnp.linspace(0.0, max_t, K + 1).astype(jnp.float32)
    edges_pad = jnp.zeros((224,), jnp.float32)
    edges_pad = edges_pad.at[:K + 1].set(edges)
    edges_pad = edges_pad.at[208:].set(jnp.float32(K) / max_t)

    mesh = plsc.VectorSubcoreMesh(core_axis_name="c", subcore_axis_name="s",
                                  num_cores=NC, num_subcores=NS)
    run = pl.kernel(
        _sc_body,
        out_type=jax.ShapeDtypeStruct((NW, 64), jnp.float32),
        mesh=mesh,
        compiler_params=pltpu.CompilerParams(needs_layout_passes=False),
        scratch_types=[
            pltpu.VMEM((K, RPW), jnp.float32),
            pltpu.VMEM((224,), jnp.float32),
            pltpu.VMEM((RPW,), jnp.float32),
            pltpu.VMEM((RPW,), jnp.int32),
            pltpu.VMEM((64,), jnp.float32),
            pltpu.SemaphoreType.DMA,
            pltpu.SemaphoreType.DMA,
        ],
    )
    parts = run(pred.T, edges_pad, dur, ev)

    sum_s = jnp.sum(parts[:, 0:16])
    sum_h = jnp.sum(parts[:, 16:32])
    cond = (jnp.min(parts[:, 32:48]) > 0.5) & (jnp.min(parts[:, 48:64]) > 0.5)
    return -jnp.where(cond, sum_s, sum_h) / B
